# Initial kernel scaffold; baseline (speedup 1.0000x reference)
import jax
import jax.numpy as jnp
from jax.experimental import pallas as pl


def kernel(x, edge_index, edge_attr, eigvecs, eigvals, params):
    raise NotImplementedError("write your pallas kernel here")


# speedup vs baseline: 1.0000x; 1.0000x over previous
import jax
import jax.numpy as jnp
from jax.experimental import pallas as pl

N = 10000
E = 320000
HID = 128
HEADS = 8
DH = HID // HEADS
PE = 16


def _ln(x, g, b):
    m = x.mean(-1, keepdims=True)
    v = ((x - m) ** 2).mean(-1, keepdims=True)
    return (x - m) / jnp.sqrt(v + 1e-5) * g + b


def _ident_body(x_ref, o_ref):
    o_ref[...] = x_ref[...]


def _ident(x):
    return pl.pallas_call(
        _ident_body, out_shape=jax.ShapeDtypeStruct(x.shape, x.dtype))(x)


def kernel(x, edge_index, edge_attr, eigvecs, eigvals, params):
    p = params
    h = _ln(x @ p['node_W'] + p['node_b'], p['in_g'], p['in_b'])
    ev = eigvecs[:, :PE]
    el = eigvals[:, :PE] + p['pe_eps']
    pe_in = jnp.stack((ev, el), axis=2)
    pe_in = jnp.where(jnp.isnan(pe_in), 0.0, pe_in)
    t = jax.nn.relu(pe_in @ p['phi1']) @ p['phi2']
    t = t.sum(1)
    pe = jax.nn.relu(t @ p['rho1_W'] + p['rho1_b']) @ p['rho2_W'] + p['rho2_b']
    h = _ident(h + pe)

    src = edge_index[0]
    dst = edge_index[1]
    for lp in p['layers']:
        q = (h @ lp['Wq'] + lp['bq']).reshape(N, HEADS, DH)
        k = (h @ lp['Wk'] + lp['bk']).reshape(N, HEADS, DH)
        v = (h @ lp['Wv'] + lp['bv']).reshape(N, HEADS, DH)
        e = (edge_attr @ lp['We']).reshape(E, HEADS, DH)
        kj = k[src] + e
        vj = v[src] + e
        qi = q[dst]
        a = (qi * kj).sum(-1) / jnp.sqrt(float(DH))
        amax = jax.ops.segment_max(a, dst, num_segments=N)
        amax = jnp.where(jnp.isfinite(amax), amax, 0.0)
        ex = jnp.exp(a - amax[dst])
        den = jax.ops.segment_sum(ex, dst, num_segments=N)
        alpha = ex / (den[dst] + 1e-16)
        out = jax.ops.segment_sum(alpha[:, :, None] * vj, dst, num_segments=N)
        out = out.reshape(N, HID)
        out = out + h @ lp['Wskip'] + lp['bskip']
        out = _ln(jax.nn.gelu(out, approximate=False), lp['ng'], lp['nb'])
        h = h + out
    y = jax.nn.gelu(h @ p['h1_W'] + p['h1_b'], approximate=False)
    y = _ln(y, p['h_g'], p['h_b'])
    return y @ p['h2_W'] + p['h2_b']


# speedup vs baseline: 6.5768x; 6.5768x over previous
import functools

import jax
import jax.numpy as jnp
import numpy as np
from jax import lax
from jax.experimental import pallas as pl
from jax.experimental.pallas import tpu as pltpu
from jax.experimental.pallas import tpu_sc as plsc

N = 10000
E = 320000
IN_DIM = 128
HID = 128
HEADS = 8
DH = HID // HEADS
PE = 16
EDIM = 16
NCLS = 40

NC = 2
NS = 16
NW = NC * NS
PER_W = E // NW
C = 80
CHUNKS = PER_W // C
AW = HID + 2 * HEADS
HALF = 5120
HROWS = 5248
ROWS_PER_TILE = HROWS // NS
PER_T = E // NS
CHUNKS_T = PER_T // C

NB = 1000
NGRID = N // NB
EB = 4000
EGRID = E // EB

_SQ = 1.0 / np.sqrt(float(DH))


def _ln_rows(x, g, b):
  m = jnp.mean(x, axis=-1, keepdims=True)
  v = jnp.mean((x - m) ** 2, axis=-1, keepdims=True)
  return (x - m) * jax.lax.rsqrt(v + 1e-5) * g + b


def _gelu_exact(x):
  return 0.5 * x * (1.0 + lax.erf(x * np.float32(1.0 / np.sqrt(2.0))))


def _pre_body(x_ref, ev_ref, el_ref, nw_ref, nb_ref, ing_ref, inb_ref,
              phi1_ref, phi2_ref, r1w_ref, r1b_ref, r2w_ref, r2b_ref,
              eps_ref, o_ref):
  x = x_ref[...]
  h = _ln_rows(x @ nw_ref[...] + nb_ref[...], ing_ref[...], inb_ref[...])
  ev = ev_ref[...]
  el = el_ref[...] + eps_ref[...]
  ev = jnp.where(jnp.isnan(ev), 0.0, ev)
  el = jnp.where(jnp.isnan(el), 0.0, el)
  phi1a = phi1_ref[0:1, :]
  phi1b = phi1_ref[1:2, :]
  t = jnp.zeros((x.shape[0], HID), jnp.float32)
  for p in range(PE):
    t += jax.nn.relu(ev[:, p:p + 1] * phi1a + el[:, p:p + 1] * phi1b)
  u = t @ phi2_ref[...]
  pe = jax.nn.relu(u @ r1w_ref[...] + r1b_ref[...]) @ r2w_ref[...] + r2b_ref[...]
  o_ref[...] = h + pe


def _qkv_body(h_ref, wq_ref, bq_ref, wk_ref, bk_ref, wv_ref, bv_ref,
              q_ref, k_ref, v_ref):
  h = h_ref[...]
  q_ref[...] = h @ wq_ref[...] + bq_ref[...]
  k_ref[...] = h @ wk_ref[...] + bk_ref[...]
  v_ref[...] = h @ wv_ref[...] + bv_ref[...]


def _eproj_body(ea_ref, we_ref, o_ref):
  o_ref[...] = ea_ref[...] @ we_ref[...]


def _post_body(p_ref, h_ref, s_ref, wskip_ref, bskip_ref,
               ng_ref, nb_ref, o_ref):
  pp = p_ref[...]
  agg = pp[:, :HID]
  den8 = pp[:, HID:HID + HEADS]
  den = den8 @ s_ref[...]
  h = h_ref[...]
  out = agg / (den + 1e-16) + h @ wskip_ref[...] + bskip_ref[...]
  out = _ln_rows(_gelu_exact(out), ng_ref[...], nb_ref[...])
  o_ref[...] = h + out


def _head_body(h_ref, w1_ref, b1_ref, hg_ref, hb_ref, w2_ref, b2_ref, o_ref):
  y = _gelu_exact(h_ref[...] @ w1_ref[...] + b1_ref[...])
  y = _ln_rows(y, hg_ref[...], hb_ref[...])
  o_ref[...] = y @ w2_ref[...] + b2_ref[...]


def _nspec(r, c):
  return pl.BlockSpec((NB, c), lambda i: (i, 0)) if r is None else r


def _wspec(rows, cols):
  return pl.BlockSpec((rows, cols), lambda i: (0, 0))


def _pre_call(x, ev, el, p):
  specs = [
      pl.BlockSpec((NB, IN_DIM), lambda i: (i, 0)),
      pl.BlockSpec((NB, PE), lambda i: (i, 0)),
      pl.BlockSpec((NB, PE), lambda i: (i, 0)),
      _wspec(IN_DIM, HID), _wspec(1, HID), _wspec(1, HID), _wspec(1, HID),
      _wspec(2, HID), _wspec(HID, HID),
      _wspec(HID, HID), _wspec(1, HID), _wspec(HID, HID), _wspec(1, HID),
      _wspec(1, PE),
  ]
  return pl.pallas_call(
      _pre_body,
      grid=(NGRID,),
      in_specs=specs,
      out_specs=pl.BlockSpec((NB, HID), lambda i: (i, 0)),
      out_shape=jax.ShapeDtypeStruct((N, HID), jnp.float32),
  )(x, ev, el, p['node_W'], p['node_b'].reshape(1, HID),
    p['in_g'].reshape(1, HID), p['in_b'].reshape(1, HID),
    p['phi1'], p['phi2'],
    p['rho1_W'], p['rho1_b'].reshape(1, HID),
    p['rho2_W'], p['rho2_b'].reshape(1, HID),
    p['pe_eps'].reshape(1, PE))


def _qkv_call(h, lp):
  specs = [
      pl.BlockSpec((NB, HID), lambda i: (i, 0)),
      _wspec(HID, HID), _wspec(1, HID),
      _wspec(HID, HID), _wspec(1, HID),
      _wspec(HID, HID), _wspec(1, HID),
  ]
  out = jax.ShapeDtypeStruct((N, HID), jnp.float32)
  return pl.pallas_call(
      _qkv_body,
      grid=(NGRID,),
      in_specs=specs,
      out_specs=[pl.BlockSpec((NB, HID), lambda i: (i, 0))] * 3,
      out_shape=[out, out, out],
  )(h, lp['Wq'], lp['bq'].reshape(1, HID),
    lp['Wk'], lp['bk'].reshape(1, HID),
    lp['Wv'], lp['bv'].reshape(1, HID))


def _eproj_call(ea, we):
  return pl.pallas_call(
      _eproj_body,
      grid=(EGRID,),
      in_specs=[pl.BlockSpec((EB, EDIM), lambda i: (i, 0)),
                _wspec(EDIM, HID)],
      out_specs=pl.BlockSpec((EB, HID), lambda i: (i, 0)),
      out_shape=jax.ShapeDtypeStruct((E, HID), jnp.float32),
  )(ea, we)


def _post_call(part, h, sel, lp):
  specs = [
      pl.BlockSpec((NB, AW), lambda i: (i, 0)),
      pl.BlockSpec((NB, HID), lambda i: (i, 0)),
      _wspec(HEADS, HID),
      _wspec(HID, HID), _wspec(1, HID), _wspec(1, HID), _wspec(1, HID),
  ]
  return pl.pallas_call(
      _post_body,
      grid=(NGRID,),
      in_specs=specs,
      out_specs=pl.BlockSpec((NB, HID), lambda i: (i, 0)),
      out_shape=jax.ShapeDtypeStruct((N, HID), jnp.float32),
  )(part, h, sel,
    lp['Wskip'], lp['bskip'].reshape(1, HID),
    lp['ng'].reshape(1, HID), lp['nb'].reshape(1, HID))


def _head_call(h, p):
  specs = [
      pl.BlockSpec((NB, HID), lambda i: (i, 0)),
      _wspec(HID, HID), _wspec(1, HID), _wspec(1, HID), _wspec(1, HID),
      _wspec(HID, NCLS), _wspec(1, NCLS),
  ]
  return pl.pallas_call(
      _head_body,
      grid=(NGRID,),
      in_specs=specs,
      out_specs=pl.BlockSpec((NB, NCLS), lambda i: (i, 0)),
      out_shape=jax.ShapeDtypeStruct((N, NCLS), jnp.float32),
  )(h, p['h1_W'], p['h1_b'].reshape(1, HID),
    p['h_g'].reshape(1, HID), p['h_b'].reshape(1, HID),
    p['h2_W'], p['h2_b'].reshape(1, NCLS))


def _edge_sc(q_hbm, k_hbm, v_hbm, e_hbm, src_hbm, dst_hbm, z_hbm, out_hbm,
             src_v, dst_v, idx_v, q_rows, k_rows, v_rows, e_rows, vals, acc,
             sem):
  cid = lax.axis_index("c")
  sid = lax.axis_index("s")

  rsl = pl.ds(sid * ROWS_PER_TILE, ROWS_PER_TILE)
  pltpu.sync_copy(z_hbm.at[rsl], acc.at[rsl])
  plsc.subcore_barrier()

  lanes = lax.iota(jnp.int32, 16)
  lo = cid * HALF

  def chunk_body(c, carry):
    base = sid * PER_T + c * C
    esl = pl.ds(base, C)
    pltpu.sync_copy(src_hbm.at[esl], src_v)
    pltpu.sync_copy(dst_hbm.at[esl], dst_v)
    for j in range(C // 16):
      jsl = pl.ds(j * 16, 16)
      il = dst_v[jsl] - lo
      inr = (il >= 0) & (il < HALF)
      idx_v[jsl] = jnp.where(inr, il, HALF)
    pltpu.async_copy(q_hbm.at[dst_v], q_rows, sem).wait()
    pltpu.async_copy(k_hbm.at[src_v], k_rows, sem).wait()
    pltpu.async_copy(v_hbm.at[src_v], v_rows, sem).wait()
    pltpu.sync_copy(e_hbm.at[esl], e_rows)

    def edge_body(i, carry2):
      exrow = jnp.zeros((16,), jnp.float32)
      for hh in range(HEADS):
        dsl = pl.ds(hh * DH, DH)
        ev = e_rows[i, dsl]
        qv = q_rows[i, dsl]
        kv = k_rows[i, dsl] + ev
        s = jnp.sum(qv * kv) * np.float32(_SQ)
        exv = jnp.exp(jnp.full((16,), s, jnp.float32))
        vals[i, dsl] = (v_rows[i, dsl] + ev) * exv
        exrow = jnp.where(lanes == hh, exv, exrow)
      vals[i, pl.ds(HID, 16)] = exrow
      return carry2

    lax.fori_loop(0, C, edge_body, 0)
    pltpu.sync_copy(vals, acc.at[idx_v], add=True)
    return carry

  lax.fori_loop(0, CHUNKS_T, chunk_body, 0)
  plsc.subcore_barrier()
  pltpu.sync_copy(acc.at[rsl], out_hbm.at[cid, rsl])


_edge_call = functools.partial(
    pl.kernel,
    mesh=plsc.VectorSubcoreMesh(core_axis_name="c", subcore_axis_name="s"),
    compiler_params=pltpu.CompilerParams(needs_layout_passes=False,
                                         use_tc_tiling_on_sc=False),
    out_type=jax.ShapeDtypeStruct((NC, HROWS, AW), jnp.float32),
    scratch_types=[
        pltpu.VMEM((C,), jnp.int32),
        pltpu.VMEM((C,), jnp.int32),
        pltpu.VMEM((C,), jnp.int32),
        pltpu.VMEM((C, HID), jnp.float32),
        pltpu.VMEM((C, HID), jnp.float32),
        pltpu.VMEM((C, HID), jnp.float32),
        pltpu.VMEM((C, HID), jnp.float32),
        pltpu.VMEM((C, AW), jnp.float32),
        pltpu.VMEM_SHARED((HROWS, AW), jnp.float32),
        pltpu.SemaphoreType.DMA,
    ],
)(_edge_sc)


def kernel(x, edge_index, edge_attr, eigvecs, eigvals, params):
  p = params
  src = edge_index[0]
  dst = edge_index[1]
  zeros_acc = jnp.zeros((HROWS, AW), jnp.float32)
  sel = jnp.asarray(np.kron(np.eye(HEADS, dtype=np.float32),
                            np.ones((1, DH), np.float32)))

  h = _pre_call(x, eigvecs[:, :PE], eigvals[:, :PE], p)
  for lp in p['layers']:
    q, k, v = _qkv_call(h, lp)
    e = _eproj_call(edge_attr, lp['We'])
    parts = _edge_call(q, k, v, e, src, dst, zeros_acc)
    part = parts[:, :HALF, :].reshape(2 * HALF, AW)[:N]
    h = _post_call(part, h, sel, lp)
  return _head_call(h, p)


# speedup vs baseline: 7.0967x; 1.0790x over previous
import functools

import jax
import jax.numpy as jnp
import numpy as np
from jax import lax
from jax.experimental import pallas as pl
from jax.experimental.pallas import tpu as pltpu
from jax.experimental.pallas import tpu_sc as plsc

N = 10000
E = 320000
IN_DIM = 128
HID = 128
HEADS = 8
DH = HID // HEADS
PE = 16
EDIM = 16
NCLS = 40

NC = 2
NS = 16
NW = NC * NS
PER_W = E // NW
C = 80
AW = HID + 2 * HEADS
HALF = 5120
HROWS = 5248
ROWS_PER_TILE = HROWS // NS
PER_T = E // NS
CHUNKS_T = PER_T // C

NB = 1000
NGRID = N // NB
EB = 4000
EGRID = E // EB

_SQ = 1.0 / np.sqrt(float(DH))


def _ln_rows(x, g, b):
  m = jnp.mean(x, axis=-1, keepdims=True)
  v = jnp.mean((x - m) ** 2, axis=-1, keepdims=True)
  return (x - m) * jax.lax.rsqrt(v + 1e-5) * g + b


def _gelu_exact(x):
  return 0.5 * x * (1.0 + lax.erf(x * np.float32(1.0 / np.sqrt(2.0))))


def _pre_body(x_ref, ev_ref, el_ref, nw_ref, nb_ref, ing_ref, inb_ref,
              phi1_ref, phi2_ref, r1w_ref, r1b_ref, r2w_ref, r2b_ref,
              eps_ref, o_ref):
  x = x_ref[...]
  h = _ln_rows(x @ nw_ref[...] + nb_ref[...], ing_ref[...], inb_ref[...])
  ev = ev_ref[...]
  el = el_ref[...] + eps_ref[...]
  ev = jnp.where(jnp.isnan(ev), 0.0, ev)
  el = jnp.where(jnp.isnan(el), 0.0, el)
  phi1a = phi1_ref[0:1, :]
  phi1b = phi1_ref[1:2, :]
  t = jnp.zeros((x.shape[0], HID), jnp.float32)
  for p in range(PE):
    t += jax.nn.relu(ev[:, p:p + 1] * phi1a + el[:, p:p + 1] * phi1b)
  u = t @ phi2_ref[...]
  pe = jax.nn.relu(u @ r1w_ref[...] + r1b_ref[...]) @ r2w_ref[...] + r2b_ref[...]
  o_ref[...] = h + pe


def _qkv_body(h_ref, wq_ref, bq_ref, wk_ref, bk_ref, wv_ref, bv_ref,
              q_ref, k_ref, v_ref):
  h = h_ref[...]
  q_ref[...] = h @ wq_ref[...] + bq_ref[...]
  k_ref[...] = h @ wk_ref[...] + bk_ref[...]
  v_ref[...] = h @ wv_ref[...] + bv_ref[...]


def _eproj_body(ea_ref, we_ref, o_ref):
  o_ref[...] = ea_ref[...] @ we_ref[...]


def _post_body(p_ref, h_ref, s_ref, wskip_ref, bskip_ref,
               ng_ref, nb_ref, o_ref):
  pp = p_ref[...]
  agg = pp[:, :HID]
  den8 = pp[:, HID:HID + HEADS]
  den = den8 @ s_ref[...]
  h = h_ref[...]
  out = agg / (den + 1e-16) + h @ wskip_ref[...] + bskip_ref[...]
  out = _ln_rows(_gelu_exact(out), ng_ref[...], nb_ref[...])
  o_ref[...] = h + out


def _head_body(h_ref, w1_ref, b1_ref, hg_ref, hb_ref, w2_ref, b2_ref, o_ref):
  y = _gelu_exact(h_ref[...] @ w1_ref[...] + b1_ref[...])
  y = _ln_rows(y, hg_ref[...], hb_ref[...])
  o_ref[...] = y @ w2_ref[...] + b2_ref[...]


def _nspec(r, c):
  return pl.BlockSpec((NB, c), lambda i: (i, 0)) if r is None else r


def _wspec(rows, cols):
  return pl.BlockSpec((rows, cols), lambda i: (0, 0))


def _pre_call(x, ev, el, p):
  specs = [
      pl.BlockSpec((NB, IN_DIM), lambda i: (i, 0)),
      pl.BlockSpec((NB, PE), lambda i: (i, 0)),
      pl.BlockSpec((NB, PE), lambda i: (i, 0)),
      _wspec(IN_DIM, HID), _wspec(1, HID), _wspec(1, HID), _wspec(1, HID),
      _wspec(2, HID), _wspec(HID, HID),
      _wspec(HID, HID), _wspec(1, HID), _wspec(HID, HID), _wspec(1, HID),
      _wspec(1, PE),
  ]
  return pl.pallas_call(
      _pre_body,
      grid=(NGRID,),
      in_specs=specs,
      out_specs=pl.BlockSpec((NB, HID), lambda i: (i, 0)),
      out_shape=jax.ShapeDtypeStruct((N, HID), jnp.float32),
  )(x, ev, el, p['node_W'], p['node_b'].reshape(1, HID),
    p['in_g'].reshape(1, HID), p['in_b'].reshape(1, HID),
    p['phi1'], p['phi2'],
    p['rho1_W'], p['rho1_b'].reshape(1, HID),
    p['rho2_W'], p['rho2_b'].reshape(1, HID),
    p['pe_eps'].reshape(1, PE))


def _qkv_call(h, lp):
  specs = [
      pl.BlockSpec((NB, HID), lambda i: (i, 0)),
      _wspec(HID, HID), _wspec(1, HID),
      _wspec(HID, HID), _wspec(1, HID),
      _wspec(HID, HID), _wspec(1, HID),
  ]
  out = jax.ShapeDtypeStruct((N, HID), jnp.float32)
  return pl.pallas_call(
      _qkv_body,
      grid=(NGRID,),
      in_specs=specs,
      out_specs=[pl.BlockSpec((NB, HID), lambda i: (i, 0))] * 3,
      out_shape=[out, out, out],
  )(h, lp['Wq'], lp['bq'].reshape(1, HID),
    lp['Wk'], lp['bk'].reshape(1, HID),
    lp['Wv'], lp['bv'].reshape(1, HID))


def _eproj_call(ea, we):
  return pl.pallas_call(
      _eproj_body,
      grid=(EGRID,),
      in_specs=[pl.BlockSpec((EB, EDIM), lambda i: (i, 0)),
                _wspec(EDIM, HID)],
      out_specs=pl.BlockSpec((EB, HID), lambda i: (i, 0)),
      out_shape=jax.ShapeDtypeStruct((E, HID), jnp.float32),
  )(ea, we)


def _post_call(part, h, sel, lp):
  specs = [
      pl.BlockSpec((NB, AW), lambda i: (i, 0)),
      pl.BlockSpec((NB, HID), lambda i: (i, 0)),
      _wspec(HEADS, HID),
      _wspec(HID, HID), _wspec(1, HID), _wspec(1, HID), _wspec(1, HID),
  ]
  return pl.pallas_call(
      _post_body,
      grid=(NGRID,),
      in_specs=specs,
      out_specs=pl.BlockSpec((NB, HID), lambda i: (i, 0)),
      out_shape=jax.ShapeDtypeStruct((N, HID), jnp.float32),
  )(part, h, sel,
    lp['Wskip'], lp['bskip'].reshape(1, HID),
    lp['ng'].reshape(1, HID), lp['nb'].reshape(1, HID))


def _head_call(h, p):
  specs = [
      pl.BlockSpec((NB, HID), lambda i: (i, 0)),
      _wspec(HID, HID), _wspec(1, HID), _wspec(1, HID), _wspec(1, HID),
      _wspec(HID, NCLS), _wspec(1, NCLS),
  ]
  return pl.pallas_call(
      _head_body,
      grid=(NGRID,),
      in_specs=specs,
      out_specs=pl.BlockSpec((NB, NCLS), lambda i: (i, 0)),
      out_shape=jax.ShapeDtypeStruct((N, NCLS), jnp.float32),
  )(h, p['h1_W'], p['h1_b'].reshape(1, HID),
    p['h_g'].reshape(1, HID), p['h_b'].reshape(1, HID),
    p['h2_W'], p['h2_b'].reshape(1, NCLS))


def _edge_sc(q_hbm, k_hbm, v_hbm, e_hbm, src_hbm, dst_hbm, z_hbm, out_hbm,
             src_v, dst_v, idx_v, q_rows, k_rows, v_rows, e_rows, vals, acc,
             sem):
  cid = lax.axis_index("c")
  sid = lax.axis_index("s")

  rsl = pl.ds(sid * ROWS_PER_TILE, ROWS_PER_TILE)
  pltpu.sync_copy(z_hbm.at[rsl], acc.at[rsl])
  plsc.subcore_barrier()

  lanes = lax.iota(jnp.int32, 16)
  lo = cid * HALF

  def chunk_body(c, carry):
    base = sid * PER_T + c * C
    esl = pl.ds(base, C)
    pltpu.sync_copy(src_hbm.at[esl], src_v)
    pltpu.sync_copy(dst_hbm.at[esl], dst_v)
    cpq = pltpu.async_copy(q_hbm.at[dst_v], q_rows, sem)
    cpk = pltpu.async_copy(k_hbm.at[src_v], k_rows, sem)
    cpv = pltpu.async_copy(v_hbm.at[src_v], v_rows, sem)
    cpe = pltpu.async_copy(e_hbm.at[esl], e_rows, sem)
    for j in range(C // 16):
      jsl = pl.ds(j * 16, 16)
      il = dst_v[jsl] - lo
      inr = (il >= 0) & (il < HALF)
      idx_v[jsl] = jnp.where(inr, il, HALF)
    cpq.wait()
    cpk.wait()
    cpv.wait()
    cpe.wait()

    def edge_body(i, carry2):
      exrow = jnp.zeros((16,), jnp.float32)
      for hh in range(HEADS):
        dsl = pl.ds(hh * DH, DH)
        ev = e_rows[i, dsl]
        qv = q_rows[i, dsl]
        kv = k_rows[i, dsl] + ev
        s = jnp.sum(qv * kv) * np.float32(_SQ)
        exv = jnp.exp(jnp.full((16,), s, jnp.float32))
        vals[i, dsl] = (v_rows[i, dsl] + ev) * exv
        exrow = jnp.where(lanes == hh, exv, exrow)
      vals[i, pl.ds(HID, 16)] = exrow
      return carry2

    lax.fori_loop(0, C, edge_body, 0)
    pltpu.sync_copy(vals, acc.at[idx_v], add=True)
    return carry

  lax.fori_loop(0, CHUNKS_T, chunk_body, 0)
  plsc.subcore_barrier()
  pltpu.sync_copy(acc.at[rsl], out_hbm.at[cid, rsl])


_edge_call = functools.partial(
    pl.kernel,
    mesh=plsc.VectorSubcoreMesh(core_axis_name="c", subcore_axis_name="s"),
    compiler_params=pltpu.CompilerParams(needs_layout_passes=False,
                                         use_tc_tiling_on_sc=False),
    out_type=jax.ShapeDtypeStruct((NC, HROWS, AW), jnp.float32),
    scratch_types=[
        pltpu.VMEM((C,), jnp.int32),
        pltpu.VMEM((C,), jnp.int32),
        pltpu.VMEM((C,), jnp.int32),
        pltpu.VMEM((C, HID), jnp.float32),
        pltpu.VMEM((C, HID), jnp.float32),
        pltpu.VMEM((C, HID), jnp.float32),
        pltpu.VMEM((C, HID), jnp.float32),
        pltpu.VMEM((C, AW), jnp.float32),
        pltpu.VMEM_SHARED((HROWS, AW), jnp.float32),
        pltpu.SemaphoreType.DMA,
    ],
)(_edge_sc)


def kernel(x, edge_index, edge_attr, eigvecs, eigvals, params):
  p = params
  src = edge_index[0]
  dst = edge_index[1]
  zeros_acc = jnp.zeros((HROWS, AW), jnp.float32)
  sel = jnp.asarray(np.kron(np.eye(HEADS, dtype=np.float32),
                            np.ones((1, DH), np.float32)))

  h = _pre_call(x, eigvecs[:, :PE], eigvals[:, :PE], p)
  for lp in p['layers']:
    q, k, v = _qkv_call(h, lp)
    e = _eproj_call(edge_attr, lp['We'])
    parts = _edge_call(q, k, v, e, src, dst, zeros_acc)
    part = parts[:, :HALF, :].reshape(2 * HALF, AW)[:N]
    h = _post_call(part, h, sel, lp)
  return _head_call(h, p)


# speedup vs baseline: 7.1032x; 1.0009x over previous
import functools

import jax
import jax.numpy as jnp
import numpy as np
from jax import lax
from jax.experimental import pallas as pl
from jax.experimental.pallas import tpu as pltpu
from jax.experimental.pallas import tpu_sc as plsc

N = 10000
E = 320000
IN_DIM = 128
HID = 128
HEADS = 8
DH = HID // HEADS
PE = 16
EDIM = 16
NCLS = 40

NC = 2
NS = 16
NW = NC * NS
PER_W = E // NW
C = 80
AW = HID + 2 * HEADS
HALF = 5120
HROWS = 5248
ROWS_PER_TILE = HROWS // NS
PER_T = E // NS
CHUNKS_T = PER_T // C

NB = 1000
NGRID = N // NB
EB = 4000
EGRID = E // EB

_SQ = 1.0 / np.sqrt(float(DH))


def _ln_rows(x, g, b):
  m = jnp.mean(x, axis=-1, keepdims=True)
  v = jnp.mean((x - m) ** 2, axis=-1, keepdims=True)
  return (x - m) * jax.lax.rsqrt(v + 1e-5) * g + b


def _gelu_exact(x):
  return 0.5 * x * (1.0 + lax.erf(x * np.float32(1.0 / np.sqrt(2.0))))


def _pre_body(x_ref, ev_ref, el_ref, nw_ref, nb_ref, ing_ref, inb_ref,
              phi1_ref, phi2_ref, r1w_ref, r1b_ref, r2w_ref, r2b_ref,
              eps_ref, o_ref):
  x = x_ref[...]
  h = _ln_rows(x @ nw_ref[...] + nb_ref[...], ing_ref[...], inb_ref[...])
  ev = ev_ref[...]
  el = el_ref[...] + eps_ref[...]
  ev = jnp.where(jnp.isnan(ev), 0.0, ev)
  el = jnp.where(jnp.isnan(el), 0.0, el)
  phi1a = phi1_ref[0:1, :]
  phi1b = phi1_ref[1:2, :]
  t = jnp.zeros((x.shape[0], HID), jnp.float32)
  for p in range(PE):
    t += jax.nn.relu(ev[:, p:p + 1] * phi1a + el[:, p:p + 1] * phi1b)
  u = t @ phi2_ref[...]
  pe = jax.nn.relu(u @ r1w_ref[...] + r1b_ref[...]) @ r2w_ref[...] + r2b_ref[...]
  o_ref[...] = h + pe


def _qkv_body(h_ref, wq_ref, bq_ref, wk_ref, bk_ref, wv_ref, bv_ref,
              q_ref, k_ref, v_ref):
  h = h_ref[...]
  q_ref[...] = h @ wq_ref[...] + bq_ref[...]
  k_ref[...] = h @ wk_ref[...] + bk_ref[...]
  v_ref[...] = h @ wv_ref[...] + bv_ref[...]


def _eproj_body(ea_ref, we_ref, o_ref):
  o_ref[...] = ea_ref[...] @ we_ref[...]


def _post_body(p_ref, h_ref, s_ref, wskip_ref, bskip_ref,
               ng_ref, nb_ref, o_ref):
  pp = p_ref[...]
  agg = pp[:, :HID]
  den8 = pp[:, HID:HID + HEADS]
  den = den8 @ s_ref[...]
  h = h_ref[...]
  out = agg / (den + 1e-16) + h @ wskip_ref[...] + bskip_ref[...]
  out = _ln_rows(_gelu_exact(out), ng_ref[...], nb_ref[...])
  o_ref[...] = h + out


def _head_body(h_ref, w1_ref, b1_ref, hg_ref, hb_ref, w2_ref, b2_ref, o_ref):
  y = _gelu_exact(h_ref[...] @ w1_ref[...] + b1_ref[...])
  y = _ln_rows(y, hg_ref[...], hb_ref[...])
  o_ref[...] = y @ w2_ref[...] + b2_ref[...]


def _nspec(r, c):
  return pl.BlockSpec((NB, c), lambda i: (i, 0)) if r is None else r


def _wspec(rows, cols):
  return pl.BlockSpec((rows, cols), lambda i: (0, 0))


def _pre_call(x, ev, el, p):
  specs = [
      pl.BlockSpec((NB, IN_DIM), lambda i: (i, 0)),
      pl.BlockSpec((NB, PE), lambda i: (i, 0)),
      pl.BlockSpec((NB, PE), lambda i: (i, 0)),
      _wspec(IN_DIM, HID), _wspec(1, HID), _wspec(1, HID), _wspec(1, HID),
      _wspec(2, HID), _wspec(HID, HID),
      _wspec(HID, HID), _wspec(1, HID), _wspec(HID, HID), _wspec(1, HID),
      _wspec(1, PE),
  ]
  return pl.pallas_call(
      _pre_body,
      grid=(NGRID,),
      in_specs=specs,
      out_specs=pl.BlockSpec((NB, HID), lambda i: (i, 0)),
      out_shape=jax.ShapeDtypeStruct((N, HID), jnp.float32),
  )(x, ev, el, p['node_W'], p['node_b'].reshape(1, HID),
    p['in_g'].reshape(1, HID), p['in_b'].reshape(1, HID),
    p['phi1'], p['phi2'],
    p['rho1_W'], p['rho1_b'].reshape(1, HID),
    p['rho2_W'], p['rho2_b'].reshape(1, HID),
    p['pe_eps'].reshape(1, PE))


def _qkv_call(h, lp):
  specs = [
      pl.BlockSpec((NB, HID), lambda i: (i, 0)),
      _wspec(HID, HID), _wspec(1, HID),
      _wspec(HID, HID), _wspec(1, HID),
      _wspec(HID, HID), _wspec(1, HID),
  ]
  out = jax.ShapeDtypeStruct((N, HID), jnp.float32)
  return pl.pallas_call(
      _qkv_body,
      grid=(NGRID,),
      in_specs=specs,
      out_specs=[pl.BlockSpec((NB, HID), lambda i: (i, 0))] * 3,
      out_shape=[out, out, out],
  )(h, lp['Wq'], lp['bq'].reshape(1, HID),
    lp['Wk'], lp['bk'].reshape(1, HID),
    lp['Wv'], lp['bv'].reshape(1, HID))


def _eproj_call(ea, we):
  return pl.pallas_call(
      _eproj_body,
      grid=(EGRID,),
      in_specs=[pl.BlockSpec((EB, EDIM), lambda i: (i, 0)),
                _wspec(EDIM, HID)],
      out_specs=pl.BlockSpec((EB, HID), lambda i: (i, 0)),
      out_shape=jax.ShapeDtypeStruct((E, HID), jnp.float32),
  )(ea, we)


def _post_call(part, h, sel, lp):
  specs = [
      pl.BlockSpec((NB, AW), lambda i: (i, 0)),
      pl.BlockSpec((NB, HID), lambda i: (i, 0)),
      _wspec(HEADS, HID),
      _wspec(HID, HID), _wspec(1, HID), _wspec(1, HID), _wspec(1, HID),
  ]
  return pl.pallas_call(
      _post_body,
      grid=(NGRID,),
      in_specs=specs,
      out_specs=pl.BlockSpec((NB, HID), lambda i: (i, 0)),
      out_shape=jax.ShapeDtypeStruct((N, HID), jnp.float32),
  )(part, h, sel,
    lp['Wskip'], lp['bskip'].reshape(1, HID),
    lp['ng'].reshape(1, HID), lp['nb'].reshape(1, HID))


def _head_call(h, p):
  specs = [
      pl.BlockSpec((NB, HID), lambda i: (i, 0)),
      _wspec(HID, HID), _wspec(1, HID), _wspec(1, HID), _wspec(1, HID),
      _wspec(HID, NCLS), _wspec(1, NCLS),
  ]
  return pl.pallas_call(
      _head_body,
      grid=(NGRID,),
      in_specs=specs,
      out_specs=pl.BlockSpec((NB, NCLS), lambda i: (i, 0)),
      out_shape=jax.ShapeDtypeStruct((N, NCLS), jnp.float32),
  )(h, p['h1_W'], p['h1_b'].reshape(1, HID),
    p['h_g'].reshape(1, HID), p['h_b'].reshape(1, HID),
    p['h2_W'], p['h2_b'].reshape(1, NCLS))


def _edge_sc(q_hbm, k_hbm, v_hbm, e_hbm, src_hbm, dst_hbm, z_hbm, out_hbm,
             src_v, dst_v, idx_v, q_rows, k_rows, v_rows, e_rows, vals, acc,
             sem):
  cid = lax.axis_index("c")
  sid = lax.axis_index("s")

  rsl = pl.ds(sid * ROWS_PER_TILE, ROWS_PER_TILE)
  pltpu.sync_copy(z_hbm.at[rsl], acc.at[rsl])
  plsc.subcore_barrier()

  lanes = lax.iota(jnp.int32, 16)
  lo = cid * HALF

  def chunk_body(c, carry):
    base = sid * PER_T + c * C
    esl = pl.ds(base, C)
    pltpu.sync_copy(src_hbm.at[esl], src_v)
    pltpu.sync_copy(dst_hbm.at[esl], dst_v)
    cpq = pltpu.async_copy(q_hbm.at[dst_v], q_rows, sem)
    cpk = pltpu.async_copy(k_hbm.at[src_v], k_rows, sem)
    cpv = pltpu.async_copy(v_hbm.at[src_v], v_rows, sem)
    cpe = pltpu.async_copy(e_hbm.at[esl], e_rows, sem)
    for j in range(C // 16):
      jsl = pl.ds(j * 16, 16)
      il = dst_v[jsl] - lo
      inr = (il >= 0) & (il < HALF)
      idx_v[jsl] = jnp.where(inr, il, HALF)
    cpq.wait()
    cpk.wait()
    cpv.wait()
    cpe.wait()

    def edge_body(ii, carry2):
      for u in range(4):
        i = ii * 4 + u
        exrow = jnp.zeros((16,), jnp.float32)
        for hh in range(HEADS):
          dsl = pl.ds(hh * DH, DH)
          ev = e_rows[i, dsl]
          qv = q_rows[i, dsl]
          kv = k_rows[i, dsl] + ev
          s = jnp.sum(qv * kv) * np.float32(_SQ)
          exv = jnp.exp(jnp.full((16,), s, jnp.float32))
          vals[i, dsl] = (v_rows[i, dsl] + ev) * exv
          exrow = jnp.where(lanes == hh, exv, exrow)
        vals[i, pl.ds(HID, 16)] = exrow
      return carry2

    lax.fori_loop(0, C // 4, edge_body, 0)
    pltpu.sync_copy(vals, acc.at[idx_v], add=True)
    return carry

  lax.fori_loop(0, CHUNKS_T, chunk_body, 0)
  plsc.subcore_barrier()
  pltpu.sync_copy(acc.at[rsl], out_hbm.at[cid, rsl])


_edge_call = functools.partial(
    pl.kernel,
    mesh=plsc.VectorSubcoreMesh(core_axis_name="c", subcore_axis_name="s"),
    compiler_params=pltpu.CompilerParams(needs_layout_passes=False,
                                         use_tc_tiling_on_sc=False),
    out_type=jax.ShapeDtypeStruct((NC, HROWS, AW), jnp.float32),
    scratch_types=[
        pltpu.VMEM((C,), jnp.int32),
        pltpu.VMEM((C,), jnp.int32),
        pltpu.VMEM((C,), jnp.int32),
        pltpu.VMEM((C, HID), jnp.float32),
        pltpu.VMEM((C, HID), jnp.float32),
        pltpu.VMEM((C, HID), jnp.float32),
        pltpu.VMEM((C, HID), jnp.float32),
        pltpu.VMEM((C, AW), jnp.float32),
        pltpu.VMEM_SHARED((HROWS, AW), jnp.float32),
        pltpu.SemaphoreType.DMA,
    ],
)(_edge_sc)


def kernel(x, edge_index, edge_attr, eigvecs, eigvals, params):
  p = params
  src = edge_index[0]
  dst = edge_index[1]
  zeros_acc = jnp.zeros((HROWS, AW), jnp.float32)
  sel = jnp.asarray(np.kron(np.eye(HEADS, dtype=np.float32),
                            np.ones((1, DH), np.float32)))

  h = _pre_call(x, eigvecs[:, :PE], eigvals[:, :PE], p)
  for lp in p['layers']:
    q, k, v = _qkv_call(h, lp)
    e = _eproj_call(edge_attr, lp['We'])
    parts = _edge_call(q, k, v, e, src, dst, zeros_acc)
    part = parts[:, :HALF, :].reshape(2 * HALF, AW)[:N]
    h = _post_call(part, h, sel, lp)
  return _head_call(h, p)
